# Initial kernel scaffold; baseline (speedup 1.0000x reference)
import jax
import jax.numpy as jnp
from jax.experimental import pallas as pl


def kernel(x, edge_index, edge_attr, Wr1, br1, Wr2, br2, Wm1, bm1, Wm2, bm2, Wa1, ba1, Wa2, ba2, Wu1, bu1, Wu2, bu2, Ws1, bs1, Ws2, bs2, gamma, beta):
    raise NotImplementedError("write your pallas kernel here")


# speedup vs baseline: 3.6997x; 3.6997x over previous
import dataclasses
import functools

import jax
import jax.numpy as jnp
from jax import lax
from jax.experimental import pallas as pl
from jax.experimental.pallas import tpu as pltpu
from jax.experimental.pallas import tpu_sc as plsc

N = 10000
E = 320000
D = 128
ED = 16
H = 256
GW = 128
BE = 1280
BN = 2000
SC_CHUNK = 128
NUM_CHUNKS = E // SC_CHUNK
NPAD = 10240
NPT = NPAD // 16
NW = 32


def _sc_gather(x, idx_flat):
    mesh = plsc.VectorSubcoreMesh(core_axis_name="c", subcore_axis_name="s")

    @functools.partial(
        pl.kernel,
        out_type=(jax.ShapeDtypeStruct((E, D), jnp.float32),
                  jax.ShapeDtypeStruct((E, D), jnp.float32)),
        mesh=mesh,
    )
    def k(x_hbm, i_hbm, xs_hbm, xd_hbm):
        def body(i_vmem, o_vmem):
            pltpu.sync_copy(x_hbm.at[i_vmem.at[0]], o_vmem)

        for half, out in ((0, xs_hbm), (1, xd_hbm)):
            pltpu.emit_pipeline(
                body,
                grid=(E // GW,),
                in_specs=[pl.BlockSpec(
                    (1, GW), index_map=lambda i, h=half: (0, i + h * (E // GW)))],
                out_specs=[pl.BlockSpec((GW, D), index_map=lambda i: (i, 0))],
                core_axis_name=("c", "s"),
                dimension_semantics=(pltpu.PARALLEL,),
            )(i_hbm, out)

    return k(x, idx_flat)


def _mlp_body(xd_ref, xs_ref, ea_ref, W1ds_ref, W1e_ref, b1_ref,
              Wm2_ref, bm2_ref, Wr2p_ref, Wr2pT_ref, br2b_ref,
              wmsg_ref, rel8_ref):
    xds = jnp.concatenate([xd_ref[...], xs_ref[...]], axis=1)
    h = jnp.dot(xds, W1ds_ref[...], preferred_element_type=jnp.float32)
    h = h + jnp.dot(ea_ref[...], W1e_ref[...], preferred_element_type=jnp.float32)
    h = jnp.maximum(h + b1_ref[...], 0.0)
    hm = h[:, :H]
    hr = h[:, H:]
    msg = jnp.dot(hm, Wm2_ref[...], preferred_element_type=jnp.float32) + bm2_ref[...]
    br2s = br2b_ref[...][:, 0:1]
    relz = jnp.dot(hr, Wr2p_ref[...], preferred_element_type=jnp.float32)[:, 0:1]
    rel_col = jax.nn.sigmoid(relz + br2s)
    wmsg_ref[...] = msg * (0.25 * rel_col)
    relz_row = lax.dot_general(Wr2pT_ref[...], hr,
                               (((1,), (1,)), ((), ())),
                               preferred_element_type=jnp.float32)
    rel8_ref[...] = jax.nn.sigmoid(relz_row + br2s)


def _tc_mlp(xd_a, xs_a, ea, W1ds, W1e, b1, Wm2, bm2, Wr2p, Wr2pT, br2b):
    nb = E // BE
    return pl.pallas_call(
        _mlp_body,
        grid=(nb,),
        in_specs=[
            pl.BlockSpec((BE, D), lambda i: (i, 0)),
            pl.BlockSpec((BE, D), lambda i: (i, 0)),
            pl.BlockSpec((BE, ED), lambda i: (i, 0)),
            pl.BlockSpec((2 * D, 2 * H), lambda i: (0, 0)),
            pl.BlockSpec((ED, 2 * H), lambda i: (0, 0)),
            pl.BlockSpec((1, 2 * H), lambda i: (0, 0)),
            pl.BlockSpec((H, D), lambda i: (0, 0)),
            pl.BlockSpec((1, D), lambda i: (0, 0)),
            pl.BlockSpec((H, 8), lambda i: (0, 0)),
            pl.BlockSpec((8, H), lambda i: (0, 0)),
            pl.BlockSpec((1, 8), lambda i: (0, 0)),
        ],
        out_specs=[
            pl.BlockSpec((BE, D), lambda i: (i, 0)),
            pl.BlockSpec((8, BE), lambda i: (0, i)),
        ],
        out_shape=[
            jax.ShapeDtypeStruct((E, D), jnp.float32),
            jax.ShapeDtypeStruct((8, E), jnp.float32),
        ],
    )(xd_a, xs_a, ea, W1ds, W1e, b1, Wm2, bm2, Wr2p, Wr2pT, br2b)


def _sc_scatter(wmsg, dst_flat, rel8, zeros_init):
    mesh = plsc.VectorSubcoreMesh(core_axis_name="c", subcore_axis_name="s")
    cp = pltpu.CompilerParams()
    if "needs_layout_passes" in pltpu.CompilerParams.__dataclass_fields__:
        cp = dataclasses.replace(cp, needs_layout_passes=False)

    @functools.partial(
        pl.kernel,
        compiler_params=cp,
        out_type=(jax.ShapeDtypeStruct((2, NPAD, D), jnp.float32),
                  jax.ShapeDtypeStruct((NW, NPAD), jnp.float32)),
        mesh=mesh,
        scratch_types=[
            pltpu.VMEM_SHARED((NPAD, D), jnp.float32),
            pltpu.VMEM((SC_CHUNK,), jnp.int32),
            pltpu.VMEM((SC_CHUNK, D), jnp.float32),
            pltpu.VMEM((SC_CHUNK,), jnp.float32),
            pltpu.VMEM((NPAD,), jnp.float32),
        ],
    )
    def k(wmsg_hbm, dst_hbm, rel_hbm, z_hbm, agg_hbm, cnt_hbm,
          acc, idx_v, val_v, rel_v, cnt_local):
        c = lax.axis_index("c")
        s = lax.axis_index("s")
        w = c * 16 + s
        pltpu.sync_copy(z_hbm.at[pl.ds(s * NPT, NPT)],
                        acc.at[pl.ds(s * NPT, NPT)])

        z16 = jnp.zeros((16,), jnp.float32)

        @pl.loop(0, NPAD // 16)
        def _(r):
            cnt_local[pl.ds(r * 16, 16)] = z16

        plsc.subcore_barrier()

        @pl.loop(0, (NUM_CHUNKS + NW - 1) // NW)
        def _(j):
            kk = j * NW + w

            @pl.when(kk < NUM_CHUNKS)
            def _():
                pltpu.sync_copy(dst_hbm.at[0, pl.ds(kk * SC_CHUNK, SC_CHUNK)],
                                idx_v)
                pltpu.sync_copy(wmsg_hbm.at[pl.ds(kk * SC_CHUNK, SC_CHUNK)],
                                val_v)
                pltpu.sync_copy(rel_hbm.at[0, pl.ds(kk * SC_CHUNK, SC_CHUNK)],
                                rel_v)
                pltpu.sync_copy(val_v, acc.at[idx_v], add=True)
                for t in range(SC_CHUNK // 16):
                    idx16 = idx_v[pl.ds(t * 16, 16)]
                    r16 = rel_v[pl.ds(t * 16, 16)]
                    plsc.addupdate_scatter(cnt_local, [idx16], r16)

        plsc.subcore_barrier()
        pltpu.sync_copy(acc.at[pl.ds(s * NPT, NPT)],
                        agg_hbm.at[c, pl.ds(s * NPT, NPT)])
        pltpu.sync_copy(cnt_local, cnt_hbm.at[w])

    return k(wmsg, dst_flat, rel8, zeros_init)


def _update_body(x_ref, a0_ref, a1_ref, cntT_ref, W1b_ref, b1b_ref,
                 W2b_ref, b2b_ref, gamma_ref, beta_ref, out_ref):
    xb = x_ref[...]
    agg = a0_ref[0] + a1_ref[0]
    cnt = jnp.sum(cntT_ref[...], axis=1, keepdims=True)
    iso = jnp.exp(-cnt)
    xa = jnp.concatenate([xb, agg], axis=1)
    h = jnp.maximum(jnp.dot(xa, W1b_ref[...],
                            preferred_element_type=jnp.float32) + b1b_ref[...],
                    0.0)
    o2 = jnp.dot(h, W2b_ref[...],
                 preferred_element_type=jnp.float32) + b2b_ref[...]
    upd = o2[:, :D]
    si = o2[:, D:]
    y = xb + (1.0 - iso) * upd + iso * si
    mu = jnp.mean(y, axis=1, keepdims=True)
    yc = y - mu
    var = jnp.mean(yc * yc, axis=1, keepdims=True)
    out_ref[...] = yc * lax.rsqrt(var + 1e-5) * gamma_ref[...] + beta_ref[...]


def _tc_update(x, aggp, cntT, W1b, b1b, W2b, b2b, gamma, beta):
    nb = N // BN
    return pl.pallas_call(
        _update_body,
        grid=(nb,),
        in_specs=[
            pl.BlockSpec((BN, D), lambda i: (i, 0)),
            pl.BlockSpec((1, BN, D), lambda i: (0, i, 0)),
            pl.BlockSpec((1, BN, D), lambda i: (1, i, 0)),
            pl.BlockSpec((BN, NW), lambda i: (i, 0)),
            pl.BlockSpec((2 * D, 2 * H), lambda i: (0, 0)),
            pl.BlockSpec((1, 2 * H), lambda i: (0, 0)),
            pl.BlockSpec((2 * H, 2 * D), lambda i: (0, 0)),
            pl.BlockSpec((1, 2 * D), lambda i: (0, 0)),
            pl.BlockSpec((1, D), lambda i: (0, 0)),
            pl.BlockSpec((1, D), lambda i: (0, 0)),
        ],
        out_specs=pl.BlockSpec((BN, D), lambda i: (i, 0)),
        out_shape=jax.ShapeDtypeStruct((N, D), jnp.float32),
    )(x, aggp, aggp, cntT, W1b, b1b, W2b, b2b, gamma, beta)


def kernel(x, edge_index, edge_attr, Wr1, br1, Wr2, br2, Wm1, bm1, Wm2, bm2,
           Wa1, ba1, Wa2, ba2, Wu1, bu1, Wu2, bu2, Ws1, bs1, Ws2, bs2,
           gamma, beta):
    f32 = jnp.float32
    W1ds = jnp.concatenate([
        jnp.concatenate([Wm1[:D], Wr1[D:2 * D]], axis=1),
        jnp.concatenate([Wm1[D:2 * D], Wr1[:D]], axis=1),
    ], axis=0)
    W1e = jnp.concatenate([Wm1[2 * D:], Wr1[2 * D:]], axis=1)
    b1 = jnp.concatenate([bm1, br1])[None, :]
    Wr2p = jnp.pad(Wr2, ((0, 0), (0, 7)))
    Wr2pT = jnp.asarray(Wr2p.T)
    br2b = jnp.broadcast_to(br2[None, :], (1, 8)) + jnp.zeros((1, 8), f32)
    bm2r = bm2[None, :]
    W1b = jnp.concatenate([
        jnp.concatenate([Wu1[:D], Ws1], axis=1),
        jnp.concatenate([Wu1[D:], jnp.zeros((D, H), f32)], axis=1),
    ], axis=0)
    b1b = jnp.concatenate([bu1, bs1])[None, :]
    W2b = jnp.concatenate([
        jnp.concatenate([Wu2, jnp.zeros((H, D), f32)], axis=1),
        jnp.concatenate([jnp.zeros((H, D), f32), Ws2], axis=1),
    ], axis=0)
    b2b = jnp.concatenate([bu2, bs2])[None, :]

    idx_flat = edge_index.reshape(1, 2 * E)
    dst_flat = edge_index[1:2]
    zeros_init = jnp.zeros((NPAD, D), f32)

    xs_a, xd_a = _sc_gather(x, idx_flat)
    wmsg, rel8 = _tc_mlp(xd_a, xs_a, edge_attr, W1ds, W1e, b1, Wm2, bm2r,
                         Wr2p, Wr2pT, br2b)
    aggp, cnt32 = _sc_scatter(wmsg, dst_flat, rel8, zeros_init)
    cntT = jnp.transpose(cnt32)
    out = _tc_update(x, aggp, cntT, W1b, b1b, W2b, b2b,
                     gamma[None, :], beta[None, :])
    rel = rel8[0]
    return (out, rel)


# speedup vs baseline: 4.4709x; 1.2085x over previous
import dataclasses
import functools

import jax
import jax.numpy as jnp
from jax import lax
from jax.experimental import pallas as pl
from jax.experimental.pallas import tpu as pltpu
from jax.experimental.pallas import tpu_sc as plsc

N = 10000
E = 320000
D = 128
ED = 16
H = 256
GW = 128
BE = 1280
BN = 2000
SC_CHUNK = 128
NUM_CHUNKS = E // SC_CHUNK
NPAD = 10240
NPT = NPAD // 16
NW = 32


def _sc_gather(x, idx_flat):
    mesh = plsc.VectorSubcoreMesh(core_axis_name="c", subcore_axis_name="s")

    @functools.partial(
        pl.kernel,
        out_type=(jax.ShapeDtypeStruct((E, D), jnp.float32),
                  jax.ShapeDtypeStruct((E, D), jnp.float32)),
        mesh=mesh,
        scratch_types=[pltpu.VMEM_SHARED((N, D), jnp.float32)],
    )
    def k(x_hbm, i_hbm, xs_hbm, xd_hbm, x_sh):
        s = lax.axis_index("s")

        @pl.when(s < 15)
        def _():
            pltpu.sync_copy(x_hbm.at[pl.ds(s * 640, 640)],
                            x_sh.at[pl.ds(s * 640, 640)])

        @pl.when(s == 15)
        def _():
            pltpu.sync_copy(x_hbm.at[pl.ds(9600, N - 9600)],
                            x_sh.at[pl.ds(9600, N - 9600)])

        plsc.subcore_barrier()

        def body(i_vmem, o_vmem):
            pltpu.sync_copy(x_sh.at[i_vmem.at[0]], o_vmem)

        for half, out in ((0, xs_hbm), (1, xd_hbm)):
            pltpu.emit_pipeline(
                body,
                grid=(E // GW,),
                in_specs=[pl.BlockSpec(
                    (1, GW), index_map=lambda i, h=half: (0, i + h * (E // GW)))],
                out_specs=[pl.BlockSpec((GW, D), index_map=lambda i: (i, 0))],
                core_axis_name=("c", "s"),
                dimension_semantics=(pltpu.PARALLEL,),
            )(i_hbm, out)

    return k(x, idx_flat)


def _mlp_body(xd_ref, xs_ref, ea_ref, W1ds_ref, W1e_ref, b1_ref,
              Wm2_ref, bm2_ref, Wr2p_ref, Wr2pT_ref, br2b_ref,
              wmsg_ref, rel8_ref):
    xds = jnp.concatenate([xd_ref[...], xs_ref[...]], axis=1).astype(jnp.bfloat16)
    h = jnp.dot(xds, W1ds_ref[...], preferred_element_type=jnp.float32)
    h = h + jnp.dot(ea_ref[...], W1e_ref[...], preferred_element_type=jnp.float32)
    h = jnp.maximum(h + b1_ref[...], 0.0)
    hb = h.astype(jnp.bfloat16)
    hm = hb[:, :H]
    hr = hb[:, H:]
    msg = jnp.dot(hm, Wm2_ref[...], preferred_element_type=jnp.float32) + bm2_ref[...]
    br2s = br2b_ref[...][:, 0:1]
    relz = jnp.dot(hr, Wr2p_ref[...], preferred_element_type=jnp.float32)[:, 0:1]
    rel_col = jax.nn.sigmoid(relz + br2s)
    wmsg_ref[...] = msg * (0.25 * rel_col)
    relz_row = lax.dot_general(Wr2pT_ref[...], hr,
                               (((1,), (1,)), ((), ())),
                               preferred_element_type=jnp.float32)
    rel8_ref[...] = jax.nn.sigmoid(relz_row + br2s)


def _tc_mlp(xd_a, xs_a, ea, W1ds, W1e, b1, Wm2, bm2, Wr2p, Wr2pT, br2b):
    nb = E // BE
    return pl.pallas_call(
        _mlp_body,
        grid=(nb,),
        in_specs=[
            pl.BlockSpec((BE, D), lambda i: (i, 0)),
            pl.BlockSpec((BE, D), lambda i: (i, 0)),
            pl.BlockSpec((BE, ED), lambda i: (i, 0)),
            pl.BlockSpec((2 * D, 2 * H), lambda i: (0, 0)),
            pl.BlockSpec((ED, 2 * H), lambda i: (0, 0)),
            pl.BlockSpec((1, 2 * H), lambda i: (0, 0)),
            pl.BlockSpec((H, D), lambda i: (0, 0)),
            pl.BlockSpec((1, D), lambda i: (0, 0)),
            pl.BlockSpec((H, 8), lambda i: (0, 0)),
            pl.BlockSpec((8, H), lambda i: (0, 0)),
            pl.BlockSpec((1, 8), lambda i: (0, 0)),
        ],
        out_specs=[
            pl.BlockSpec((BE, D), lambda i: (i, 0)),
            pl.BlockSpec((8, BE), lambda i: (0, i)),
        ],
        out_shape=[
            jax.ShapeDtypeStruct((E, D), jnp.float32),
            jax.ShapeDtypeStruct((8, E), jnp.float32),
        ],
    )(xd_a, xs_a, ea, W1ds, W1e, b1, Wm2, bm2, Wr2p, Wr2pT, br2b)


def _sc_scatter(wmsg, dst_flat, rel8, zeros_init):
    mesh = plsc.VectorSubcoreMesh(core_axis_name="c", subcore_axis_name="s")
    cp = pltpu.CompilerParams()
    if "needs_layout_passes" in pltpu.CompilerParams.__dataclass_fields__:
        cp = dataclasses.replace(cp, needs_layout_passes=False)

    @functools.partial(
        pl.kernel,
        compiler_params=cp,
        out_type=(jax.ShapeDtypeStruct((2, NPAD, D), jnp.float32),
                  jax.ShapeDtypeStruct((NW, NPAD), jnp.float32)),
        mesh=mesh,
        scratch_types=[
            pltpu.VMEM_SHARED((NPAD, D), jnp.float32),
            pltpu.VMEM((SC_CHUNK,), jnp.int32),
            pltpu.VMEM((SC_CHUNK, D), jnp.float32),
            pltpu.VMEM((SC_CHUNK,), jnp.float32),
            pltpu.VMEM((NPAD,), jnp.float32),
        ],
    )
    def k(wmsg_hbm, dst_hbm, rel_hbm, z_hbm, agg_hbm, cnt_hbm,
          acc, idx_v, val_v, rel_v, cnt_local):
        c = lax.axis_index("c")
        s = lax.axis_index("s")
        w = c * 16 + s
        pltpu.sync_copy(z_hbm.at[pl.ds(s * NPT, NPT)],
                        acc.at[pl.ds(s * NPT, NPT)])

        z16 = jnp.zeros((16,), jnp.float32)

        @pl.loop(0, NPAD // 16)
        def _(r):
            cnt_local[pl.ds(r * 16, 16)] = z16

        plsc.subcore_barrier()

        @pl.loop(0, (NUM_CHUNKS + NW - 1) // NW)
        def _(j):
            kk = j * NW + w

            @pl.when(kk < NUM_CHUNKS)
            def _():
                pltpu.sync_copy(dst_hbm.at[0, pl.ds(kk * SC_CHUNK, SC_CHUNK)],
                                idx_v)
                pltpu.sync_copy(wmsg_hbm.at[pl.ds(kk * SC_CHUNK, SC_CHUNK)],
                                val_v)
                pltpu.sync_copy(rel_hbm.at[0, pl.ds(kk * SC_CHUNK, SC_CHUNK)],
                                rel_v)
                pltpu.sync_copy(val_v, acc.at[idx_v], add=True)
                for t in range(SC_CHUNK // 16):
                    idx16 = idx_v[pl.ds(t * 16, 16)]
                    r16 = rel_v[pl.ds(t * 16, 16)]
                    plsc.addupdate_scatter(cnt_local, [idx16], r16)

        plsc.subcore_barrier()
        pltpu.sync_copy(acc.at[pl.ds(s * NPT, NPT)],
                        agg_hbm.at[c, pl.ds(s * NPT, NPT)])
        pltpu.sync_copy(cnt_local, cnt_hbm.at[w])

    return k(wmsg, dst_flat, rel8, zeros_init)


def _update_body(x_ref, a0_ref, a1_ref, cntT_ref, W1b_ref, b1b_ref,
                 W2b_ref, b2b_ref, gamma_ref, beta_ref, out_ref):
    xb = x_ref[...]
    agg = a0_ref[0] + a1_ref[0]
    cnt = jnp.sum(cntT_ref[...], axis=1, keepdims=True)
    iso = jnp.exp(-cnt)
    xa = jnp.concatenate([xb, agg], axis=1)
    h = jnp.maximum(jnp.dot(xa, W1b_ref[...],
                            preferred_element_type=jnp.float32) + b1b_ref[...],
                    0.0)
    o2 = jnp.dot(h, W2b_ref[...],
                 preferred_element_type=jnp.float32) + b2b_ref[...]
    upd = o2[:, :D]
    si = o2[:, D:]
    y = xb + (1.0 - iso) * upd + iso * si
    mu = jnp.mean(y, axis=1, keepdims=True)
    yc = y - mu
    var = jnp.mean(yc * yc, axis=1, keepdims=True)
    out_ref[...] = yc * lax.rsqrt(var + 1e-5) * gamma_ref[...] + beta_ref[...]


def _tc_update(x, aggp, cntT, W1b, b1b, W2b, b2b, gamma, beta):
    nb = N // BN
    return pl.pallas_call(
        _update_body,
        grid=(nb,),
        in_specs=[
            pl.BlockSpec((BN, D), lambda i: (i, 0)),
            pl.BlockSpec((1, BN, D), lambda i: (0, i, 0)),
            pl.BlockSpec((1, BN, D), lambda i: (1, i, 0)),
            pl.BlockSpec((BN, NW), lambda i: (i, 0)),
            pl.BlockSpec((2 * D, 2 * H), lambda i: (0, 0)),
            pl.BlockSpec((1, 2 * H), lambda i: (0, 0)),
            pl.BlockSpec((2 * H, 2 * D), lambda i: (0, 0)),
            pl.BlockSpec((1, 2 * D), lambda i: (0, 0)),
            pl.BlockSpec((1, D), lambda i: (0, 0)),
            pl.BlockSpec((1, D), lambda i: (0, 0)),
        ],
        out_specs=pl.BlockSpec((BN, D), lambda i: (i, 0)),
        out_shape=jax.ShapeDtypeStruct((N, D), jnp.float32),
    )(x, aggp, aggp, cntT, W1b, b1b, W2b, b2b, gamma, beta)


def kernel(x, edge_index, edge_attr, Wr1, br1, Wr2, br2, Wm1, bm1, Wm2, bm2,
           Wa1, ba1, Wa2, ba2, Wu1, bu1, Wu2, bu2, Ws1, bs1, Ws2, bs2,
           gamma, beta):
    f32 = jnp.float32
    bf16 = jnp.bfloat16
    W1ds = jnp.concatenate([
        jnp.concatenate([Wm1[:D], Wr1[D:2 * D]], axis=1),
        jnp.concatenate([Wm1[D:2 * D], Wr1[:D]], axis=1),
    ], axis=0).astype(bf16)
    W1e = jnp.concatenate([Wm1[2 * D:], Wr1[2 * D:]], axis=1).astype(bf16)
    b1 = jnp.concatenate([bm1, br1])[None, :]
    Wr2p = jnp.pad(Wr2, ((0, 0), (0, 7))).astype(bf16)
    Wr2pT = jnp.asarray(Wr2p.T)
    br2b = jnp.broadcast_to(br2[None, :], (1, 8)) + jnp.zeros((1, 8), f32)
    bm2r = bm2[None, :]
    Wm2b = Wm2.astype(bf16)
    W1b = jnp.concatenate([
        jnp.concatenate([Wu1[:D], Ws1], axis=1),
        jnp.concatenate([Wu1[D:], jnp.zeros((D, H), f32)], axis=1),
    ], axis=0)
    b1b = jnp.concatenate([bu1, bs1])[None, :]
    W2b = jnp.concatenate([
        jnp.concatenate([Wu2, jnp.zeros((H, D), f32)], axis=1),
        jnp.concatenate([jnp.zeros((H, D), f32), Ws2], axis=1),
    ], axis=0)
    b2b = jnp.concatenate([bu2, bs2])[None, :]

    idx_flat = edge_index.reshape(1, 2 * E)
    dst_flat = edge_index[1:2]
    zeros_init = jnp.zeros((NPAD, D), f32)

    xs_a, xd_a = _sc_gather(x, idx_flat)
    wmsg, rel8 = _tc_mlp(xd_a, xs_a, edge_attr.astype(bf16), W1ds, W1e, b1,
                         Wm2b, bm2r, Wr2p, Wr2pT, br2b)
    aggp, cnt32 = _sc_scatter(wmsg, dst_flat, rel8, zeros_init)
    cntT = jnp.transpose(cnt32)
    out = _tc_update(x, aggp, cntT, W1b, b1b, W2b, b2b,
                     gamma[None, :], beta[None, :])
    rel = rel8[0]
    return (out, rel)


# speedup vs baseline: 5.1799x; 1.1586x over previous
import dataclasses
import functools

import jax
import jax.numpy as jnp
from jax import lax
from jax.experimental import pallas as pl
from jax.experimental.pallas import tpu as pltpu
from jax.experimental.pallas import tpu_sc as plsc

N = 10000
E = 320000
D = 128
ED = 16
H = 256
GW = 128
BE = 1280
BN = 2000
SC_CHUNK = 128
NUM_CHUNKS = E // SC_CHUNK
NPAD = 10240
NPT = NPAD // 16
NW = 32


def _sc_gather(x, idx_flat):
    mesh = plsc.VectorSubcoreMesh(core_axis_name="c", subcore_axis_name="s")

    @functools.partial(
        pl.kernel,
        out_type=(jax.ShapeDtypeStruct((E, D), jnp.float32),
                  jax.ShapeDtypeStruct((E, D), jnp.float32)),
        mesh=mesh,
        scratch_types=[pltpu.VMEM_SHARED((N, D), jnp.float32)],
    )
    def k(x_hbm, i_hbm, xs_hbm, xd_hbm, x_sh):
        s = lax.axis_index("s")

        @pl.when(s < 15)
        def _():
            pltpu.sync_copy(x_hbm.at[pl.ds(s * 640, 640)],
                            x_sh.at[pl.ds(s * 640, 640)])

        @pl.when(s == 15)
        def _():
            pltpu.sync_copy(x_hbm.at[pl.ds(9600, N - 9600)],
                            x_sh.at[pl.ds(9600, N - 9600)])

        plsc.subcore_barrier()

        def body(i_vmem, o_vmem):
            pltpu.sync_copy(x_sh.at[i_vmem.at[0]], o_vmem)

        for half, out in ((0, xs_hbm), (1, xd_hbm)):
            pltpu.emit_pipeline(
                body,
                grid=(E // GW,),
                in_specs=[pl.BlockSpec(
                    (1, GW), index_map=lambda i, h=half: (0, i + h * (E // GW)))],
                out_specs=[pl.BlockSpec((GW, D), index_map=lambda i: (i, 0))],
                core_axis_name=("c", "s"),
                dimension_semantics=(pltpu.PARALLEL,),
            )(i_hbm, out)

    return k(x, idx_flat)


def _mlp_body(xd_ref, xs_ref, ea_ref, W1ds_ref, W1e_ref, b1_ref,
              Wm2_ref, bm2_ref, Wr2p_ref, Wr2pT_ref, br2b_ref,
              wmsg_ref, rel8_ref):
    xds = jnp.concatenate([xd_ref[...], xs_ref[...]], axis=1).astype(jnp.bfloat16)
    h = jnp.dot(xds, W1ds_ref[...], preferred_element_type=jnp.float32)
    h = h + jnp.dot(ea_ref[...], W1e_ref[...], preferred_element_type=jnp.float32)
    h = jnp.maximum(h + b1_ref[...], 0.0)
    hb = h.astype(jnp.bfloat16)
    hm = hb[:, :H]
    hr = hb[:, H:]
    msg = jnp.dot(hm, Wm2_ref[...], preferred_element_type=jnp.float32) + bm2_ref[...]
    br2s = br2b_ref[...][:, 0:1]
    relz = jnp.dot(hr, Wr2p_ref[...], preferred_element_type=jnp.float32)[:, 0:1]
    rel_col = jax.nn.sigmoid(relz + br2s)
    wmsg_ref[...] = msg * (0.25 * rel_col)
    relz_row = lax.dot_general(Wr2pT_ref[...], hr,
                               (((1,), (1,)), ((), ())),
                               preferred_element_type=jnp.float32)
    rel8_ref[...] = jax.nn.sigmoid(relz_row + br2s)


def _tc_mlp(xd_a, xs_a, ea, W1ds, W1e, b1, Wm2, bm2, Wr2p, Wr2pT, br2b):
    nb = E // BE
    return pl.pallas_call(
        _mlp_body,
        grid=(nb,),
        in_specs=[
            pl.BlockSpec((BE, D), lambda i: (i, 0)),
            pl.BlockSpec((BE, D), lambda i: (i, 0)),
            pl.BlockSpec((BE, ED), lambda i: (i, 0)),
            pl.BlockSpec((2 * D, 2 * H), lambda i: (0, 0)),
            pl.BlockSpec((ED, 2 * H), lambda i: (0, 0)),
            pl.BlockSpec((1, 2 * H), lambda i: (0, 0)),
            pl.BlockSpec((H, D), lambda i: (0, 0)),
            pl.BlockSpec((1, D), lambda i: (0, 0)),
            pl.BlockSpec((H, 8), lambda i: (0, 0)),
            pl.BlockSpec((8, H), lambda i: (0, 0)),
            pl.BlockSpec((1, 8), lambda i: (0, 0)),
        ],
        out_specs=[
            pl.BlockSpec((BE, D), lambda i: (i, 0)),
            pl.BlockSpec((8, BE), lambda i: (0, i)),
        ],
        out_shape=[
            jax.ShapeDtypeStruct((E, D), jnp.float32),
            jax.ShapeDtypeStruct((8, E), jnp.float32),
        ],
    )(xd_a, xs_a, ea, W1ds, W1e, b1, Wm2, bm2, Wr2p, Wr2pT, br2b)


def _sc_scatter(wmsg, dst_flat, rel8, zeros_init):
    mesh = plsc.VectorSubcoreMesh(core_axis_name="c", subcore_axis_name="s")
    cp = pltpu.CompilerParams()
    if "needs_layout_passes" in pltpu.CompilerParams.__dataclass_fields__:
        cp = dataclasses.replace(cp, needs_layout_passes=False)

    CH = SC_CHUNK
    STEPS = (NUM_CHUNKS // NW) & ~1
    REM = NUM_CHUNKS - STEPS * NW

    @functools.partial(
        pl.kernel,
        compiler_params=cp,
        out_type=(jax.ShapeDtypeStruct((2, NPAD, D), jnp.float32),
                  jax.ShapeDtypeStruct((NW, NPAD), jnp.float32)),
        mesh=mesh,
        scratch_types=[
            pltpu.VMEM_SHARED((NPAD, D), jnp.float32),
            pltpu.VMEM((2, CH), jnp.int32),
            pltpu.VMEM((2, CH, D), jnp.float32),
            pltpu.VMEM((2, CH), jnp.float32),
            pltpu.VMEM((NPAD,), jnp.float32),
            pltpu.SemaphoreType.DMA((2,)),
        ],
    )
    def k(wmsg_hbm, dst_hbm, rel_hbm, z_hbm, agg_hbm, cnt_hbm,
          acc, idx_v, val_v, rel_v, cnt_local, lsem):
        c = lax.axis_index("c")
        s = lax.axis_index("s")
        w = c * 16 + s
        pltpu.sync_copy(z_hbm.at[pl.ds(s * NPT, NPT)],
                        acc.at[pl.ds(s * NPT, NPT)])

        z16 = jnp.zeros((16,), jnp.float32)

        @pl.loop(0, NPAD // 16)
        def _(r):
            cnt_local[pl.ds(r * 16, 16)] = z16

        plsc.subcore_barrier()

        def start_loads(kk, b):
            pltpu.async_copy(dst_hbm.at[0, pl.ds(kk * CH, CH)],
                             idx_v.at[b], lsem.at[b])
            pltpu.async_copy(rel_hbm.at[0, pl.ds(kk * CH, CH)],
                             rel_v.at[b], lsem.at[b])
            pltpu.async_copy(wmsg_hbm.at[pl.ds(kk * CH, CH)],
                             val_v.at[b], lsem.at[b])

        def wait_loads(kk, b):
            pltpu.make_async_copy(dst_hbm.at[0, pl.ds(kk * CH, CH)],
                                  idx_v.at[b], lsem.at[b]).wait()
            pltpu.make_async_copy(rel_hbm.at[0, pl.ds(kk * CH, CH)],
                                  rel_v.at[b], lsem.at[b]).wait()
            pltpu.make_async_copy(wmsg_hbm.at[pl.ds(kk * CH, CH)],
                                  val_v.at[b], lsem.at[b]).wait()

        def process(kk, b, ssem):
            h = pltpu.async_copy(val_v.at[b], acc.at[idx_v.at[b]], ssem,
                                 add=True)
            for t in range(CH // 16):
                idx16 = idx_v[b, pl.ds(t * 16, 16)]
                r16 = rel_v[b, pl.ds(t * 16, 16)]
                plsc.addupdate_scatter(cnt_local, [idx16], r16)
            h.wait()

        start_loads(w, 0)

        @pl.loop(0, STEPS, step=2)
        def _(jj):
            for b in (0, 1):
                j = jj + b
                kk = j * NW + w
                wait_loads(kk, b)
                if b == 0:
                    start_loads(kk + NW, 1)
                else:
                    @pl.when(jj < STEPS - 2)
                    def _():
                        start_loads(kk + NW, 0)
                process(kk, b, lsem.at[b])

        @pl.when(w < REM)
        def _():
            kk = STEPS * NW + w
            pltpu.sync_copy(dst_hbm.at[0, pl.ds(kk * CH, CH)], idx_v.at[0])
            pltpu.sync_copy(rel_hbm.at[0, pl.ds(kk * CH, CH)], rel_v.at[0])
            pltpu.sync_copy(wmsg_hbm.at[pl.ds(kk * CH, CH)], val_v.at[0])
            process(kk, 0, lsem.at[0])

        plsc.subcore_barrier()
        pltpu.sync_copy(acc.at[pl.ds(s * NPT, NPT)],
                        agg_hbm.at[c, pl.ds(s * NPT, NPT)])
        pltpu.sync_copy(cnt_local, cnt_hbm.at[w])

    return k(wmsg, dst_flat, rel8, zeros_init)


def _update_body(x_ref, a0_ref, a1_ref, cntT_ref, W1b_ref, b1b_ref,
                 W2b_ref, b2b_ref, gamma_ref, beta_ref, out_ref):
    xb = x_ref[...]
    agg = a0_ref[0] + a1_ref[0]
    cnt = jnp.sum(cntT_ref[...], axis=1, keepdims=True)
    iso = jnp.exp(-cnt)
    xa = jnp.concatenate([xb, agg], axis=1)
    h = jnp.maximum(jnp.dot(xa, W1b_ref[...],
                            preferred_element_type=jnp.float32) + b1b_ref[...],
                    0.0)
    o2 = jnp.dot(h, W2b_ref[...],
                 preferred_element_type=jnp.float32) + b2b_ref[...]
    upd = o2[:, :D]
    si = o2[:, D:]
    y = xb + (1.0 - iso) * upd + iso * si
    mu = jnp.mean(y, axis=1, keepdims=True)
    yc = y - mu
    var = jnp.mean(yc * yc, axis=1, keepdims=True)
    out_ref[...] = yc * lax.rsqrt(var + 1e-5) * gamma_ref[...] + beta_ref[...]


def _tc_update(x, aggp, cntT, W1b, b1b, W2b, b2b, gamma, beta):
    nb = N // BN
    return pl.pallas_call(
        _update_body,
        grid=(nb,),
        in_specs=[
            pl.BlockSpec((BN, D), lambda i: (i, 0)),
            pl.BlockSpec((1, BN, D), lambda i: (0, i, 0)),
            pl.BlockSpec((1, BN, D), lambda i: (1, i, 0)),
            pl.BlockSpec((BN, NW), lambda i: (i, 0)),
            pl.BlockSpec((2 * D, 2 * H), lambda i: (0, 0)),
            pl.BlockSpec((1, 2 * H), lambda i: (0, 0)),
            pl.BlockSpec((2 * H, 2 * D), lambda i: (0, 0)),
            pl.BlockSpec((1, 2 * D), lambda i: (0, 0)),
            pl.BlockSpec((1, D), lambda i: (0, 0)),
            pl.BlockSpec((1, D), lambda i: (0, 0)),
        ],
        out_specs=pl.BlockSpec((BN, D), lambda i: (i, 0)),
        out_shape=jax.ShapeDtypeStruct((N, D), jnp.float32),
    )(x, aggp, aggp, cntT, W1b, b1b, W2b, b2b, gamma, beta)


def kernel(x, edge_index, edge_attr, Wr1, br1, Wr2, br2, Wm1, bm1, Wm2, bm2,
           Wa1, ba1, Wa2, ba2, Wu1, bu1, Wu2, bu2, Ws1, bs1, Ws2, bs2,
           gamma, beta):
    f32 = jnp.float32
    bf16 = jnp.bfloat16
    W1ds = jnp.concatenate([
        jnp.concatenate([Wm1[:D], Wr1[D:2 * D]], axis=1),
        jnp.concatenate([Wm1[D:2 * D], Wr1[:D]], axis=1),
    ], axis=0).astype(bf16)
    W1e = jnp.concatenate([Wm1[2 * D:], Wr1[2 * D:]], axis=1).astype(bf16)
    b1 = jnp.concatenate([bm1, br1])[None, :]
    Wr2p = jnp.pad(Wr2, ((0, 0), (0, 7))).astype(bf16)
    Wr2pT = jnp.asarray(Wr2p.T)
    br2b = jnp.broadcast_to(br2[None, :], (1, 8)) + jnp.zeros((1, 8), f32)
    bm2r = bm2[None, :]
    Wm2b = Wm2.astype(bf16)
    W1b = jnp.concatenate([
        jnp.concatenate([Wu1[:D], Ws1], axis=1),
        jnp.concatenate([Wu1[D:], jnp.zeros((D, H), f32)], axis=1),
    ], axis=0)
    b1b = jnp.concatenate([bu1, bs1])[None, :]
    W2b = jnp.concatenate([
        jnp.concatenate([Wu2, jnp.zeros((H, D), f32)], axis=1),
        jnp.concatenate([jnp.zeros((H, D), f32), Ws2], axis=1),
    ], axis=0)
    b2b = jnp.concatenate([bu2, bs2])[None, :]

    idx_flat = edge_index.reshape(1, 2 * E)
    dst_flat = edge_index[1:2]
    zeros_init = jnp.zeros((NPAD, D), f32)

    xs_a, xd_a = _sc_gather(x, idx_flat)
    wmsg, rel8 = _tc_mlp(xd_a, xs_a, edge_attr.astype(bf16), W1ds, W1e, b1,
                         Wm2b, bm2r, Wr2p, Wr2pT, br2b)
    aggp, cnt32 = _sc_scatter(wmsg, dst_flat, rel8, zeros_init)
    cntT = jnp.transpose(cnt32)
    out = _tc_update(x, aggp, cntT, W1b, b1b, W2b, b2b,
                     gamma[None, :], beta[None, :])
    rel = rel8[0]
    return (out, rel)


# speedup vs baseline: 5.7581x; 1.1116x over previous
import dataclasses
import functools

import jax
import jax.numpy as jnp
from jax import lax
from jax.experimental import pallas as pl
from jax.experimental.pallas import tpu as pltpu
from jax.experimental.pallas import tpu_sc as plsc

N = 10000
E = 320000
D = 128
ED = 16
H = 256
GW = 128
BE = 1280
BN = 2000
SC_CHUNK = 128
NPAD = 10240
NPT = NPAD // 16
NW = 32
NSLICES = 2
ES = E // NSLICES


def _sc_gather(x, idx_flat, off, ne):
    mesh = plsc.VectorSubcoreMesh(core_axis_name="c", subcore_axis_name="s")

    @functools.partial(
        pl.kernel,
        out_type=(jax.ShapeDtypeStruct((ne, D), jnp.float32),
                  jax.ShapeDtypeStruct((ne, D), jnp.float32)),
        mesh=mesh,
        scratch_types=[pltpu.VMEM_SHARED((N, D), jnp.float32)],
    )
    def k(x_hbm, i_hbm, xs_hbm, xd_hbm, x_sh):
        s = lax.axis_index("s")

        @pl.when(s < 15)
        def _():
            pltpu.sync_copy(x_hbm.at[pl.ds(s * 640, 640)],
                            x_sh.at[pl.ds(s * 640, 640)])

        @pl.when(s == 15)
        def _():
            pltpu.sync_copy(x_hbm.at[pl.ds(9600, N - 9600)],
                            x_sh.at[pl.ds(9600, N - 9600)])

        plsc.subcore_barrier()

        def body(i_vmem, o_vmem):
            pltpu.sync_copy(x_sh.at[i_vmem.at[0]], o_vmem)

        for base, out in ((off, xs_hbm), (E + off, xd_hbm)):
            pltpu.emit_pipeline(
                body,
                grid=(ne // GW,),
                in_specs=[pl.BlockSpec(
                    (1, GW), index_map=lambda i, b=base: (0, i + b // GW))],
                out_specs=[pl.BlockSpec((GW, D), index_map=lambda i: (i, 0))],
                core_axis_name=("c", "s"),
                dimension_semantics=(pltpu.PARALLEL,),
            )(i_hbm, out)

    return k(x, idx_flat)


def _mlp_body(xd_ref, xs_ref, ea_ref, W1ds_ref, W1e_ref, b1_ref,
              Wm2_ref, bm2_ref, Wr2p_ref, Wr2pT_ref, br2b_ref,
              wmsg_ref, rel8_ref):
    xds = jnp.concatenate([xd_ref[...], xs_ref[...]], axis=1).astype(jnp.bfloat16)
    h = jnp.dot(xds, W1ds_ref[...], preferred_element_type=jnp.float32)
    h = h + jnp.dot(ea_ref[...], W1e_ref[...], preferred_element_type=jnp.float32)
    h = jnp.maximum(h + b1_ref[...], 0.0)
    hb = h.astype(jnp.bfloat16)
    hm = hb[:, :H]
    hr = hb[:, H:]
    msg = jnp.dot(hm, Wm2_ref[...], preferred_element_type=jnp.float32) + bm2_ref[...]
    br2s = br2b_ref[...][:, 0:1]
    relz = jnp.dot(hr, Wr2p_ref[...], preferred_element_type=jnp.float32)[:, 0:1]
    rel_col = jax.nn.sigmoid(relz + br2s)
    wmsg_ref[...] = msg * (0.25 * rel_col)
    relz_row = lax.dot_general(Wr2pT_ref[...], hr,
                               (((1,), (1,)), ((), ())),
                               preferred_element_type=jnp.float32)
    rel8_ref[...] = jax.nn.sigmoid(relz_row + br2s)


def _tc_mlp(xd_a, xs_a, ea, W1ds, W1e, b1, Wm2, bm2, Wr2p, Wr2pT, br2b,
            off, ne):
    nb = ne // BE
    ob = off // BE
    return pl.pallas_call(
        _mlp_body,
        grid=(nb,),
        in_specs=[
            pl.BlockSpec((BE, D), lambda i: (i, 0)),
            pl.BlockSpec((BE, D), lambda i: (i, 0)),
            pl.BlockSpec((BE, ED), lambda i: (i + ob, 0)),
            pl.BlockSpec((2 * D, 2 * H), lambda i: (0, 0)),
            pl.BlockSpec((ED, 2 * H), lambda i: (0, 0)),
            pl.BlockSpec((1, 2 * H), lambda i: (0, 0)),
            pl.BlockSpec((H, D), lambda i: (0, 0)),
            pl.BlockSpec((1, D), lambda i: (0, 0)),
            pl.BlockSpec((H, 8), lambda i: (0, 0)),
            pl.BlockSpec((8, H), lambda i: (0, 0)),
            pl.BlockSpec((1, 8), lambda i: (0, 0)),
        ],
        out_specs=[
            pl.BlockSpec((BE, D), lambda i: (i, 0)),
            pl.BlockSpec((8, BE), lambda i: (0, i)),
        ],
        out_shape=[
            jax.ShapeDtypeStruct((ne, D), jnp.float32),
            jax.ShapeDtypeStruct((8, ne), jnp.float32),
        ],
    )(xd_a, xs_a, ea, W1ds, W1e, b1, Wm2, bm2, Wr2p, Wr2pT, br2b)


def _sc_scatter(wmsg, dst_flat, rel8, zeros_init, off, ne):
    mesh = plsc.VectorSubcoreMesh(core_axis_name="c", subcore_axis_name="s")
    cp = pltpu.CompilerParams()
    if "needs_layout_passes" in pltpu.CompilerParams.__dataclass_fields__:
        cp = dataclasses.replace(cp, needs_layout_passes=False)

    CH = SC_CHUNK
    num_chunks = ne // CH
    koff = off // CH
    STEPS = (num_chunks // NW) & ~1
    REM = num_chunks - STEPS * NW

    @functools.partial(
        pl.kernel,
        compiler_params=cp,
        out_type=(jax.ShapeDtypeStruct((2, NPAD, D), jnp.float32),
                  jax.ShapeDtypeStruct((NW, NPAD), jnp.float32)),
        mesh=mesh,
        scratch_types=[
            pltpu.VMEM_SHARED((NPAD, D), jnp.float32),
            pltpu.VMEM((2, CH), jnp.int32),
            pltpu.VMEM((2, CH, D), jnp.float32),
            pltpu.VMEM((2, CH), jnp.float32),
            pltpu.VMEM((NPAD,), jnp.float32),
            pltpu.SemaphoreType.DMA((2,)),
        ],
    )
    def k(wmsg_hbm, dst_hbm, rel_hbm, z_hbm, agg_hbm, cnt_hbm,
          acc, idx_v, val_v, rel_v, cnt_local, lsem):
        c = lax.axis_index("c")
        s = lax.axis_index("s")
        w = c * 16 + s
        pltpu.sync_copy(z_hbm.at[pl.ds(s * NPT, NPT)],
                        acc.at[pl.ds(s * NPT, NPT)])

        z16 = jnp.zeros((16,), jnp.float32)

        @pl.loop(0, NPAD // 16)
        def _(r):
            cnt_local[pl.ds(r * 16, 16)] = z16

        plsc.subcore_barrier()

        def start_loads(kk, b):
            pltpu.async_copy(dst_hbm.at[0, pl.ds((koff + kk) * CH, CH)],
                             idx_v.at[b], lsem.at[b])
            pltpu.async_copy(rel_hbm.at[0, pl.ds(kk * CH, CH)],
                             rel_v.at[b], lsem.at[b])
            pltpu.async_copy(wmsg_hbm.at[pl.ds(kk * CH, CH)],
                             val_v.at[b], lsem.at[b])

        def wait_loads(kk, b):
            pltpu.make_async_copy(dst_hbm.at[0, pl.ds((koff + kk) * CH, CH)],
                                  idx_v.at[b], lsem.at[b]).wait()
            pltpu.make_async_copy(rel_hbm.at[0, pl.ds(kk * CH, CH)],
                                  rel_v.at[b], lsem.at[b]).wait()
            pltpu.make_async_copy(wmsg_hbm.at[pl.ds(kk * CH, CH)],
                                  val_v.at[b], lsem.at[b]).wait()

        def process(b, ssem):
            h = pltpu.async_copy(val_v.at[b], acc.at[idx_v.at[b]], ssem,
                                 add=True)
            for t in range(CH // 16):
                idx16 = idx_v[b, pl.ds(t * 16, 16)]
                r16 = rel_v[b, pl.ds(t * 16, 16)]
                plsc.addupdate_scatter(cnt_local, [idx16], r16)
            h.wait()

        if STEPS > 0:
            start_loads(w, 0)

            @pl.loop(0, STEPS, step=2)
            def _(jj):
                for b in (0, 1):
                    j = jj + b
                    kk = j * NW + w
                    wait_loads(kk, b)
                    if b == 0:
                        start_loads(kk + NW, 1)
                    else:
                        @pl.when(jj < STEPS - 2)
                        def _():
                            start_loads(kk + NW, 0)
                    process(b, lsem.at[b])

        base = STEPS * NW
        while base < num_chunks:
            cnt_here = min(NW, num_chunks - base)

            @pl.when(w < cnt_here)
            def _(base=base):
                kk = base + w
                pltpu.sync_copy(dst_hbm.at[0, pl.ds((koff + kk) * CH, CH)],
                                idx_v.at[0])
                pltpu.sync_copy(rel_hbm.at[0, pl.ds(kk * CH, CH)],
                                rel_v.at[0])
                pltpu.sync_copy(wmsg_hbm.at[pl.ds(kk * CH, CH)], val_v.at[0])
                process(0, lsem.at[0])

            base += cnt_here

        plsc.subcore_barrier()
        pltpu.sync_copy(acc.at[pl.ds(s * NPT, NPT)],
                        agg_hbm.at[c, pl.ds(s * NPT, NPT)])
        pltpu.sync_copy(cnt_local, cnt_hbm.at[w])

    return k(wmsg, dst_flat, rel8, zeros_init)


def _update_body(x_ref, a0_ref, a1_ref, a2_ref, a3_ref, cntT_ref,
                 W1b_ref, b1b_ref, W2b_ref, b2b_ref, gamma_ref, beta_ref,
                 out_ref):
    xb = x_ref[...]
    agg = (a0_ref[0] + a1_ref[0]) + (a2_ref[0] + a3_ref[0])
    cnt = jnp.sum(cntT_ref[...], axis=1, keepdims=True)
    iso = jnp.exp(-cnt)
    xa = jnp.concatenate([xb, agg], axis=1)
    h = jnp.maximum(jnp.dot(xa, W1b_ref[...],
                            preferred_element_type=jnp.float32) + b1b_ref[...],
                    0.0)
    o2 = jnp.dot(h, W2b_ref[...],
                 preferred_element_type=jnp.float32) + b2b_ref[...]
    upd = o2[:, :D]
    si = o2[:, D:]
    y = xb + (1.0 - iso) * upd + iso * si
    mu = jnp.mean(y, axis=1, keepdims=True)
    yc = y - mu
    var = jnp.mean(yc * yc, axis=1, keepdims=True)
    out_ref[...] = yc * lax.rsqrt(var + 1e-5) * gamma_ref[...] + beta_ref[...]


def _tc_update(x, aggp0, aggp1, cntT, W1b, b1b, W2b, b2b, gamma, beta):
    nb = N // BN
    return pl.pallas_call(
        _update_body,
        grid=(nb,),
        in_specs=[
            pl.BlockSpec((BN, D), lambda i: (i, 0)),
            pl.BlockSpec((1, BN, D), lambda i: (0, i, 0)),
            pl.BlockSpec((1, BN, D), lambda i: (1, i, 0)),
            pl.BlockSpec((1, BN, D), lambda i: (0, i, 0)),
            pl.BlockSpec((1, BN, D), lambda i: (1, i, 0)),
            pl.BlockSpec((BN, 2 * NW), lambda i: (i, 0)),
            pl.BlockSpec((2 * D, 2 * H), lambda i: (0, 0)),
            pl.BlockSpec((1, 2 * H), lambda i: (0, 0)),
            pl.BlockSpec((2 * H, 2 * D), lambda i: (0, 0)),
            pl.BlockSpec((1, 2 * D), lambda i: (0, 0)),
            pl.BlockSpec((1, D), lambda i: (0, 0)),
            pl.BlockSpec((1, D), lambda i: (0, 0)),
        ],
        out_specs=pl.BlockSpec((BN, D), lambda i: (i, 0)),
        out_shape=jax.ShapeDtypeStruct((N, D), jnp.float32),
    )(x, aggp0, aggp0, aggp1, aggp1, cntT, W1b, b1b, W2b, b2b, gamma, beta)


def kernel(x, edge_index, edge_attr, Wr1, br1, Wr2, br2, Wm1, bm1, Wm2, bm2,
           Wa1, ba1, Wa2, ba2, Wu1, bu1, Wu2, bu2, Ws1, bs1, Ws2, bs2,
           gamma, beta):
    f32 = jnp.float32
    bf16 = jnp.bfloat16
    W1ds = jnp.concatenate([
        jnp.concatenate([Wm1[:D], Wr1[D:2 * D]], axis=1),
        jnp.concatenate([Wm1[D:2 * D], Wr1[:D]], axis=1),
    ], axis=0).astype(bf16)
    W1e = jnp.concatenate([Wm1[2 * D:], Wr1[2 * D:]], axis=1).astype(bf16)
    b1 = jnp.concatenate([bm1, br1])[None, :]
    Wr2p = jnp.pad(Wr2, ((0, 0), (0, 7))).astype(bf16)
    Wr2pT = jnp.asarray(Wr2p.T)
    br2b = jnp.broadcast_to(br2[None, :], (1, 8)) + jnp.zeros((1, 8), f32)
    bm2r = bm2[None, :]
    Wm2b = Wm2.astype(bf16)
    W1b = jnp.concatenate([
        jnp.concatenate([Wu1[:D], Ws1], axis=1),
        jnp.concatenate([Wu1[D:], jnp.zeros((D, H), f32)], axis=1),
    ], axis=0)
    b1b = jnp.concatenate([bu1, bs1])[None, :]
    W2b = jnp.concatenate([
        jnp.concatenate([Wu2, jnp.zeros((H, D), f32)], axis=1),
        jnp.concatenate([jnp.zeros((H, D), f32), Ws2], axis=1),
    ], axis=0)
    b2b = jnp.concatenate([bu2, bs2])[None, :]

    idx_flat = edge_index.reshape(1, 2 * E)
    dst_flat = edge_index[1:2]
    zeros_init = jnp.zeros((NPAD, D), f32)
    ea16 = edge_attr.astype(bf16)

    aggs, cnts, rels = [], [], []
    for h in range(NSLICES):
        off = h * ES
        xs_a, xd_a = _sc_gather(x, idx_flat, off, ES)
        wmsg, rel8 = _tc_mlp(xd_a, xs_a, ea16, W1ds, W1e, b1, Wm2b, bm2r,
                             Wr2p, Wr2pT, br2b, off, ES)
        aggp, cnt32 = _sc_scatter(wmsg, dst_flat, rel8, zeros_init, off, ES)
        aggs.append(aggp)
        cnts.append(cnt32)
        rels.append(rel8[0])

    cntT = jnp.transpose(jnp.concatenate(cnts, axis=0))
    out = _tc_update(x, aggs[0], aggs[1], cntT, W1b, b1b, W2b, b2b,
                     gamma[None, :], beta[None, :])
    rel = jnp.concatenate(rels)
    return (out, rel)


# speedup vs baseline: 5.7646x; 1.0011x over previous
import dataclasses
import functools

import jax
import jax.numpy as jnp
from jax import lax
from jax.experimental import pallas as pl
from jax.experimental.pallas import tpu as pltpu
from jax.experimental.pallas import tpu_sc as plsc

N = 10000
E = 320000
D = 128
ED = 16
H = 256
GW = 128
BE = 1280
BN = 2000
SC_CHUNK = 128
NPAD = 10240
NPT = NPAD // 16
NW = 32
NSLICES = 2
ES = E // NSLICES


def _sc_gather(x, idx_flat, off, ne):
    mesh = plsc.VectorSubcoreMesh(core_axis_name="c", subcore_axis_name="s")

    @functools.partial(
        pl.kernel,
        out_type=(jax.ShapeDtypeStruct((ne, D), jnp.float32),
                  jax.ShapeDtypeStruct((ne, D), jnp.float32)),
        mesh=mesh,
        scratch_types=[pltpu.VMEM_SHARED((N, D), jnp.float32)],
    )
    def k(x_hbm, i_hbm, xs_hbm, xd_hbm, x_sh):
        s = lax.axis_index("s")

        @pl.when(s < 15)
        def _():
            pltpu.sync_copy(x_hbm.at[pl.ds(s * 640, 640)],
                            x_sh.at[pl.ds(s * 640, 640)])

        @pl.when(s == 15)
        def _():
            pltpu.sync_copy(x_hbm.at[pl.ds(9600, N - 9600)],
                            x_sh.at[pl.ds(9600, N - 9600)])

        plsc.subcore_barrier()

        def body(i_vmem, o_vmem):
            pltpu.sync_copy(x_sh.at[i_vmem.at[0]], o_vmem)

        for base, out in ((off, xs_hbm), (E + off, xd_hbm)):
            pltpu.emit_pipeline(
                body,
                grid=(ne // GW,),
                in_specs=[pl.BlockSpec(
                    (1, GW), index_map=lambda i, b=base: (0, i + b // GW))],
                out_specs=[pl.BlockSpec((GW, D), index_map=lambda i: (i, 0))],
                core_axis_name=("c", "s"),
                dimension_semantics=(pltpu.PARALLEL,),
            )(i_hbm, out)

    return k(x, idx_flat)


def _mlp_body(xd_ref, xs_ref, ea_ref, W1ds_ref, W1e_ref, b1_ref,
              Wm2_ref, bm2_ref, Wr2p_ref, Wr2pT_ref, br2b_ref,
              wmsg_ref, rel8_ref):
    xds = jnp.concatenate([xd_ref[...], xs_ref[...]], axis=1).astype(jnp.bfloat16)
    h = jnp.dot(xds, W1ds_ref[...], preferred_element_type=jnp.float32)
    h = h + jnp.dot(ea_ref[...], W1e_ref[...], preferred_element_type=jnp.float32)
    h = jnp.maximum(h + b1_ref[...], 0.0)
    hb = h.astype(jnp.bfloat16)
    hm = hb[:, :H]
    hr = hb[:, H:]
    msg = jnp.dot(hm, Wm2_ref[...], preferred_element_type=jnp.float32) + bm2_ref[...]
    br2s = br2b_ref[...][:, 0:1]
    relz = jnp.dot(hr, Wr2p_ref[...], preferred_element_type=jnp.float32)[:, 0:1]
    rel_col = jax.nn.sigmoid(relz + br2s)
    wmsg_ref[...] = msg * (0.25 * rel_col)
    relz_row = lax.dot_general(Wr2pT_ref[...], hr,
                               (((1,), (1,)), ((), ())),
                               preferred_element_type=jnp.float32)
    rel8_ref[...] = jax.nn.sigmoid(relz_row + br2s)


def _tc_mlp(xd_a, xs_a, ea, W1ds, W1e, b1, Wm2, bm2, Wr2p, Wr2pT, br2b,
            off, ne):
    nb = ne // BE
    ob = off // BE
    return pl.pallas_call(
        _mlp_body,
        grid=(nb,),
        in_specs=[
            pl.BlockSpec((BE, D), lambda i: (i, 0)),
            pl.BlockSpec((BE, D), lambda i: (i, 0)),
            pl.BlockSpec((BE, ED), lambda i: (i + ob, 0)),
            pl.BlockSpec((2 * D, 2 * H), lambda i: (0, 0)),
            pl.BlockSpec((ED, 2 * H), lambda i: (0, 0)),
            pl.BlockSpec((1, 2 * H), lambda i: (0, 0)),
            pl.BlockSpec((H, D), lambda i: (0, 0)),
            pl.BlockSpec((1, D), lambda i: (0, 0)),
            pl.BlockSpec((H, 8), lambda i: (0, 0)),
            pl.BlockSpec((8, H), lambda i: (0, 0)),
            pl.BlockSpec((1, 8), lambda i: (0, 0)),
        ],
        out_specs=[
            pl.BlockSpec((BE, D), lambda i: (i, 0)),
            pl.BlockSpec((8, BE), lambda i: (0, i)),
        ],
        out_shape=[
            jax.ShapeDtypeStruct((ne, D), jnp.float32),
            jax.ShapeDtypeStruct((8, ne), jnp.float32),
        ],
    )(xd_a, xs_a, ea, W1ds, W1e, b1, Wm2, bm2, Wr2p, Wr2pT, br2b)


def _sc_scatter(wmsg, dst_flat, rel8, zeros_init, off, ne):
    mesh = plsc.VectorSubcoreMesh(core_axis_name="c", subcore_axis_name="s")
    cp = pltpu.CompilerParams()
    if "needs_layout_passes" in pltpu.CompilerParams.__dataclass_fields__:
        cp = dataclasses.replace(cp, needs_layout_passes=False)

    CH = SC_CHUNK
    num_chunks = ne // CH
    koff = off // CH
    STEPS = (num_chunks // NW) & ~1
    REM = num_chunks - STEPS * NW

    @functools.partial(
        pl.kernel,
        compiler_params=cp,
        out_type=(jax.ShapeDtypeStruct((2, NPAD, D), jnp.float32),
                  jax.ShapeDtypeStruct((NW, NPAD), jnp.float32)),
        mesh=mesh,
        scratch_types=[
            pltpu.VMEM_SHARED((NPAD, D), jnp.float32),
            pltpu.VMEM((2, CH), jnp.int32),
            pltpu.VMEM((2, CH, D), jnp.float32),
            pltpu.VMEM((2, CH), jnp.float32),
            pltpu.VMEM((NPAD,), jnp.float32),
            pltpu.SemaphoreType.DMA((2,)),
        ],
    )
    def k(wmsg_hbm, dst_hbm, rel_hbm, z_hbm, agg_hbm, cnt_hbm,
          acc, idx_v, val_v, rel_v, cnt_local, lsem):
        c = lax.axis_index("c")
        s = lax.axis_index("s")
        w = c * 16 + s
        pltpu.sync_copy(z_hbm.at[pl.ds(s * NPT, NPT)],
                        acc.at[pl.ds(s * NPT, NPT)])

        z16 = jnp.zeros((16,), jnp.float32)

        @pl.loop(0, NPAD // 16)
        def _(r):
            cnt_local[pl.ds(r * 16, 16)] = z16

        plsc.subcore_barrier()

        def start_loads(kk, b):
            pltpu.async_copy(dst_hbm.at[0, pl.ds((koff + kk) * CH, CH)],
                             idx_v.at[b], lsem.at[b])
            pltpu.async_copy(rel_hbm.at[0, pl.ds(kk * CH, CH)],
                             rel_v.at[b], lsem.at[b])
            pltpu.async_copy(wmsg_hbm.at[pl.ds(kk * CH, CH)],
                             val_v.at[b], lsem.at[b])

        def wait_loads(kk, b):
            pltpu.make_async_copy(dst_hbm.at[0, pl.ds((koff + kk) * CH, CH)],
                                  idx_v.at[b], lsem.at[b]).wait()
            pltpu.make_async_copy(rel_hbm.at[0, pl.ds(kk * CH, CH)],
                                  rel_v.at[b], lsem.at[b]).wait()
            pltpu.make_async_copy(wmsg_hbm.at[pl.ds(kk * CH, CH)],
                                  val_v.at[b], lsem.at[b]).wait()

        def process(b, ssem):
            h = pltpu.async_copy(val_v.at[b], acc.at[idx_v.at[b]], ssem,
                                 add=True)
            for t in range(CH // 16):
                idx16 = idx_v[b, pl.ds(t * 16, 16)]
                r16 = rel_v[b, pl.ds(t * 16, 16)]
                plsc.addupdate_scatter(cnt_local, [idx16], r16)
            h.wait()

        if STEPS > 0:
            start_loads(w, 0)

            @pl.loop(0, STEPS, step=2)
            def _(jj):
                for b in (0, 1):
                    j = jj + b
                    kk = j * NW + w
                    wait_loads(kk, b)
                    if b == 0:
                        start_loads(kk + NW, 1)
                    else:
                        @pl.when(jj < STEPS - 2)
                        def _():
                            start_loads(kk + NW, 0)
                    process(b, lsem.at[b])

        base = STEPS * NW
        while base < num_chunks:
            cnt_here = min(NW, num_chunks - base)

            @pl.when(w < cnt_here)
            def _(base=base):
                kk = base + w
                pltpu.sync_copy(dst_hbm.at[0, pl.ds((koff + kk) * CH, CH)],
                                idx_v.at[0])
                pltpu.sync_copy(rel_hbm.at[0, pl.ds(kk * CH, CH)],
                                rel_v.at[0])
                pltpu.sync_copy(wmsg_hbm.at[pl.ds(kk * CH, CH)], val_v.at[0])
                process(0, lsem.at[0])

            base += cnt_here

        plsc.subcore_barrier()
        pltpu.sync_copy(acc.at[pl.ds(s * NPT, NPT)],
                        agg_hbm.at[c, pl.ds(s * NPT, NPT)])
        pltpu.sync_copy(cnt_local, cnt_hbm.at[w])

    return k(wmsg, dst_flat, rel8, zeros_init)


def _update_body(x_ref, a0_ref, a1_ref, a2_ref, a3_ref, cntT_ref,
                 W1b_ref, b1b_ref, W2b_ref, b2b_ref, gamma_ref, beta_ref,
                 out_ref):
    xb = x_ref[...]
    agg = (a0_ref[0] + a1_ref[0]) + (a2_ref[0] + a3_ref[0])
    cnt = jnp.sum(cntT_ref[...], axis=1, keepdims=True)
    iso = jnp.exp(-cnt)
    xa = jnp.concatenate([xb, agg], axis=1)
    h = jnp.maximum(jnp.dot(xa, W1b_ref[...],
                            preferred_element_type=jnp.float32) + b1b_ref[...],
                    0.0)
    o2 = jnp.dot(h, W2b_ref[...],
                 preferred_element_type=jnp.float32) + b2b_ref[...]
    upd = o2[:, :D]
    si = o2[:, D:]
    y = xb + (1.0 - iso) * upd + iso * si
    mu = jnp.mean(y, axis=1, keepdims=True)
    yc = y - mu
    var = jnp.mean(yc * yc, axis=1, keepdims=True)
    out_ref[...] = yc * lax.rsqrt(var + 1e-5) * gamma_ref[...] + beta_ref[...]


def _tc_update(x, aggp0, aggp1, cntT, W1b, b1b, W2b, b2b, gamma, beta):
    nb = N // BN
    return pl.pallas_call(
        _update_body,
        grid=(nb,),
        in_specs=[
            pl.BlockSpec((BN, D), lambda i: (i, 0)),
            pl.BlockSpec((1, BN, D), lambda i: (0, i, 0)),
            pl.BlockSpec((1, BN, D), lambda i: (1, i, 0)),
            pl.BlockSpec((1, BN, D), lambda i: (0, i, 0)),
            pl.BlockSpec((1, BN, D), lambda i: (1, i, 0)),
            pl.BlockSpec((BN, 2 * NW), lambda i: (i, 0)),
            pl.BlockSpec((2 * D, 2 * H), lambda i: (0, 0)),
            pl.BlockSpec((1, 2 * H), lambda i: (0, 0)),
            pl.BlockSpec((2 * H, 2 * D), lambda i: (0, 0)),
            pl.BlockSpec((1, 2 * D), lambda i: (0, 0)),
            pl.BlockSpec((1, D), lambda i: (0, 0)),
            pl.BlockSpec((1, D), lambda i: (0, 0)),
        ],
        out_specs=pl.BlockSpec((BN, D), lambda i: (i, 0)),
        out_shape=jax.ShapeDtypeStruct((N, D), jnp.float32),
    )(x, aggp0, aggp0, aggp1, aggp1, cntT, W1b, b1b, W2b, b2b, gamma, beta)


def kernel(x, edge_index, edge_attr, Wr1, br1, Wr2, br2, Wm1, bm1, Wm2, bm2,
           Wa1, ba1, Wa2, ba2, Wu1, bu1, Wu2, bu2, Ws1, bs1, Ws2, bs2,
           gamma, beta):
    f32 = jnp.float32
    bf16 = jnp.bfloat16
    W1ds = jnp.concatenate([
        jnp.concatenate([Wm1[:D], Wr1[D:2 * D]], axis=1),
        jnp.concatenate([Wm1[D:2 * D], Wr1[:D]], axis=1),
    ], axis=0).astype(bf16)
    W1e = jnp.concatenate([Wm1[2 * D:], Wr1[2 * D:]], axis=1).astype(bf16)
    b1 = jnp.concatenate([bm1, br1])[None, :]
    Wr2p = jnp.pad(Wr2, ((0, 0), (0, 7))).astype(bf16)
    Wr2pT = jnp.asarray(Wr2p.T)
    br2b = jnp.broadcast_to(br2[None, :], (1, 8)) + jnp.zeros((1, 8), f32)
    bm2r = bm2[None, :]
    Wm2b = Wm2.astype(bf16)
    W1b = jnp.concatenate([
        jnp.concatenate([Wu1[:D], Ws1], axis=1),
        jnp.concatenate([Wu1[D:], jnp.zeros((D, H), f32)], axis=1),
    ], axis=0)
    b1b = jnp.concatenate([bu1, bs1])[None, :]
    W2b = jnp.concatenate([
        jnp.concatenate([Wu2, jnp.zeros((H, D), f32)], axis=1),
        jnp.concatenate([jnp.zeros((H, D), f32), Ws2], axis=1),
    ], axis=0)
    b2b = jnp.concatenate([bu2, bs2])[None, :]

    idx_flat = edge_index.reshape(1, 2 * E)
    dst_flat = edge_index[1:2]
    zeros_init = jnp.zeros((NPAD, D), f32)
    ea16 = edge_attr.astype(bf16)

    gath = [_sc_gather(x, idx_flat, h * ES, ES) for h in range(NSLICES)]
    mlps = [_tc_mlp(gath[h][1], gath[h][0], ea16, W1ds, W1e, b1, Wm2b, bm2r,
                    Wr2p, Wr2pT, br2b, h * ES, ES) for h in range(NSLICES)]
    scat = [_sc_scatter(mlps[h][0], dst_flat, mlps[h][1], zeros_init,
                        h * ES, ES) for h in range(NSLICES)]
    aggs = [sc[0] for sc in scat]
    cnts = [sc[1] for sc in scat]
    rels = [mlps[h][1][0] for h in range(NSLICES)]

    cntT = jnp.transpose(jnp.concatenate(cnts, axis=0))
    out = _tc_update(x, aggs[0], aggs[1], cntT, W1b, b1b, W2b, b2b,
                     gamma[None, :], beta[None, :])
    rel = jnp.concatenate(rels)
    return (out, rel)
